# Initial kernel scaffold; baseline (speedup 1.0000x reference)
import functools
import jax
import jax.numpy as jnp
from jax import lax
from jax.experimental import pallas as pl
from jax.experimental.pallas import tpu as pltpu, tpu_sc as plsc

E = 8
TOP_K = 2
CAP_F = 1.25
AUX_COEF = 0.01
B, T, C, I = 2, 2048, 1024, 4096
N = B * T
CAP = 640
NEC = E * CAP
NW = 32
XD_ROWS = 5376
SENT = NEC
BLK = 512
TI = 512

_NEG = jnp.float32(-1e30)


def _router_body(x_ref, wr_ref, sel1_ref, sel2_ref, sl1_ref, sl2_ref,
                 g1_ref, g2_ref, aux_ref, oh_ref, pos_ref):
    xf = x_ref[...]
    wr = wr_ref[...]
    logits = lax.dot_general(xf, wr, (((1,), (1,)), ((), ())))

    lane = lax.broadcasted_iota(jnp.int32, (N, E), 1)
    m1 = jnp.max(logits, axis=1, keepdims=True)
    eq1 = logits == m1
    idx1 = jnp.min(jnp.where(eq1, lane, E), axis=1, keepdims=True)
    oh1 = (lane == idx1)
    oh1f = oh1.astype(jnp.float32)
    logits2 = jnp.where(oh1, _NEG, logits)
    m2 = jnp.max(logits2, axis=1, keepdims=True)
    eq2 = logits2 == m2
    idx2 = jnp.min(jnp.where(eq2, lane, E), axis=1, keepdims=True)
    oh2 = (lane == idx2)
    oh2f = oh2.astype(jnp.float32)

    ex = jnp.exp(logits - m1)
    sex = jnp.sum(ex, axis=1, keepdims=True)
    probs = ex / sex
    gv1 = jnp.max(probs, axis=1, keepdims=True)
    gv2 = jnp.max(probs - 2.0 * oh1f, axis=1, keepdims=True)
    gsum = jnp.maximum(gv1 + gv2, 1e-9)
    gate1 = gv1 / gsum
    gate2 = gv2 / gsum

    oh_ref[...] = jnp.concatenate([oh1f, oh2f], axis=1)
    r_i = lax.broadcasted_iota(jnp.int32, (BLK, BLK), 0)
    c_i = lax.broadcasted_iota(jnp.int32, (BLK, BLK), 1)
    tri = jnp.where(r_i > c_i, 1.0, 0.0).astype(jnp.float32)

    def body(b, carry):
        blk = oh_ref[pl.ds(b * BLK, BLK), :]
        pos_blk = lax.dot_general(
            tri, blk, (((1,), (0,)), ((), ())),
            precision=lax.Precision.HIGHEST) + carry
        pos_ref[pl.ds(b * BLK, BLK), :] = pos_blk
        return carry + jnp.sum(blk, axis=0, keepdims=True)

    carry = lax.fori_loop(0, N // BLK, body,
                          jnp.zeros((1, 2 * E), jnp.float32))
    posm = pos_ref[...]
    pos1 = jnp.sum(posm[:, :E] * oh1f, axis=1, keepdims=True)
    pos2r = jnp.sum(posm[:, E:] * oh2f, axis=1, keepdims=True)
    tot1 = carry[:, :E]
    tot2 = carry[:, E:]
    count1 = jnp.minimum(tot1, float(CAP))
    pos2 = pos2r + jnp.sum(oh2f * count1, axis=1, keepdims=True)

    mask1 = pos1 < float(CAP)
    mask2 = pos2 < float(CAP)
    slot1 = idx1 * CAP + pos1.astype(jnp.int32)
    slot2 = idx2 * CAP + pos2.astype(jnp.int32)
    sl1_ref[...] = jnp.where(mask1, slot1, SENT)
    sl2_ref[...] = jnp.where(mask2, slot2, SENT)
    sel1_ref[...] = jnp.where(mask1, slot1, 0)
    sel2_ref[...] = jnp.where(mask2, slot2, 0)

    g1m = gate1 * mask1.astype(jnp.float32)
    g2m = gate2 * mask2.astype(jnp.float32)
    denom = jnp.maximum(g1m + g2m, 1e-9)
    g1_ref[...] = g1m / denom
    g2_ref[...] = g2m / denom

    imp = jnp.sum(probs, axis=0, keepdims=True) / float(N)
    loadv = (tot1 + tot2) / float(N * TOP_K)
    aux_ref[...] = jnp.sum(imp * loadv, axis=1, keepdims=True) * (E * AUX_COEF)


def _k1_router(x_flat, wr):
    i32 = lambda: jax.ShapeDtypeStruct((N, 1), jnp.int32)
    f32 = lambda: jax.ShapeDtypeStruct((N, 1), jnp.float32)
    return pl.pallas_call(
        _router_body,
        out_shape=(i32(), i32(), i32(), i32(), f32(), f32(),
                   jax.ShapeDtypeStruct((1, 1), jnp.float32)),
        scratch_shapes=[pltpu.VMEM((N, 2 * E), jnp.float32),
                        pltpu.VMEM((N, 2 * E), jnp.float32)],
    )(x_flat, wr)


def _tokmap_body(sl1_hbm, sl2_hbm, tok_hbm, sl_v, tok_v):
    wid = lax.axis_index("s") * 2 + lax.axis_index("c")

    @pl.when(wid == 0)
    def _():
        def init(i, _):
            tok_v[pl.ds(i * 16, 16)] = jnp.zeros((16,), jnp.int32)
            return 0
        lax.fori_loop(0, XD_ROWS // 16, init, 0)

        pltpu.sync_copy(sl1_hbm, sl_v)

        def scat1(i, _):
            idx = sl_v[pl.ds(i * 16, 16)]
            vals = i * 16 + lax.iota(jnp.int32, 16)
            plsc.store_scatter(tok_v, [idx], vals)
            return 0
        lax.fori_loop(0, N // 16, scat1, 0)

        pltpu.sync_copy(sl2_hbm, sl_v)

        def scat2(i, _):
            idx = sl_v[pl.ds(i * 16, 16)]
            vals = i * 16 + lax.iota(jnp.int32, 16)
            plsc.store_scatter(tok_v, [idx], vals)
            return 0
        lax.fori_loop(0, N // 16, scat2, 0)

        pltpu.sync_copy(tok_v, tok_hbm)


def _k2a_tokmap(sl1, sl2):
    mesh = plsc.VectorSubcoreMesh(core_axis_name="c", subcore_axis_name="s")
    return pl.kernel(
        _tokmap_body,
        out_type=jax.ShapeDtypeStruct((XD_ROWS,), jnp.int32),
        mesh=mesh,
        scratch_types=[pltpu.VMEM((N,), jnp.int32),
                       pltpu.VMEM((XD_ROWS,), jnp.int32)],
    )(sl1, sl2)


def _dispatch_body(x_hbm, tok_hbm, xd_hbm, idx_v, rows_v, sem):
    wid = lax.axis_index("s") * 2 + lax.axis_index("c")
    base = wid * (XD_ROWS // NW)
    for c in range(3):
        off = base + c * 56
        pltpu.sync_copy(tok_hbm.at[pl.ds(off, 56)], idx_v)
        pltpu.async_copy(x_hbm.at[idx_v], rows_v, sem).wait()
        pltpu.sync_copy(rows_v, xd_hbm.at[pl.ds(off, 56)])


def _k2b_dispatch(x_flat, tok):
    mesh = plsc.VectorSubcoreMesh(core_axis_name="c", subcore_axis_name="s")
    return pl.kernel(
        _dispatch_body,
        out_type=jax.ShapeDtypeStruct((XD_ROWS, C), jnp.float32),
        mesh=mesh,
        scratch_types=[pltpu.VMEM((56,), jnp.int32),
                       pltpu.VMEM((56, C), jnp.float32),
                       pltpu.SemaphoreType.DMA],
    )(x_flat, tok)


def _mlp_body(xd_ref, wg_ref, wu_ref, wp_ref, eo_ref):
    it = pl.program_id(1)
    xb = xd_ref[...]
    wg = wg_ref[0]
    wu = wu_ref[0]
    wp = wp_ref[0]
    g = lax.dot_general(xb, wg, (((1,), (1,)), ((), ())))
    u = lax.dot_general(xb, wu, (((1,), (1,)), ((), ())))
    h = g * (1.0 / (1.0 + jnp.exp(-g))) * u
    contrib = lax.dot_general(h, wp, (((1,), (1,)), ((), ())))

    @pl.when(it == 0)
    def _():
        eo_ref[...] = contrib

    @pl.when(it != 0)
    def _():
        eo_ref[...] += contrib


def _k3_mlp(xd, wg, wu, wp):
    grid = (E, I // TI)
    return pl.pallas_call(
        _mlp_body,
        grid=grid,
        in_specs=[
            pl.BlockSpec((CAP, C), lambda e, it: (e, 0)),
            pl.BlockSpec((1, TI, C), lambda e, it: (e, it, 0)),
            pl.BlockSpec((1, TI, C), lambda e, it: (e, it, 0)),
            pl.BlockSpec((1, C, TI), lambda e, it: (e, 0, it)),
        ],
        out_specs=pl.BlockSpec((CAP, C), lambda e, it: (e, 0)),
        out_shape=jax.ShapeDtypeStruct((NEC, C), jnp.float32),
        compiler_params=pltpu.CompilerParams(
            dimension_semantics=("parallel", "arbitrary")),
    )(xd, wg, wu, wp)


def _gather2_body(eo_hbm, s1_hbm, s2_hbm, y1_hbm, y2_hbm, idx_v, rows_v, sem):
    wid = lax.axis_index("s") * 2 + lax.axis_index("c")
    base = wid * (N // NW)
    for c in range(2):
        off = base + c * 64
        pltpu.sync_copy(s1_hbm.at[pl.ds(off, 64)], idx_v)
        pltpu.async_copy(eo_hbm.at[idx_v], rows_v, sem).wait()
        pltpu.sync_copy(rows_v, y1_hbm.at[pl.ds(off, 64)])
        pltpu.sync_copy(s2_hbm.at[pl.ds(off, 64)], idx_v)
        pltpu.async_copy(eo_hbm.at[idx_v], rows_v, sem).wait()
        pltpu.sync_copy(rows_v, y2_hbm.at[pl.ds(off, 64)])


def _k4_gather(eo, sel1, sel2):
    mesh = plsc.VectorSubcoreMesh(core_axis_name="c", subcore_axis_name="s")
    return pl.kernel(
        _gather2_body,
        out_type=(jax.ShapeDtypeStruct((N, C), jnp.float32),
                  jax.ShapeDtypeStruct((N, C), jnp.float32)),
        mesh=mesh,
        scratch_types=[pltpu.VMEM((64,), jnp.int32),
                       pltpu.VMEM((64, C), jnp.float32),
                       pltpu.SemaphoreType.DMA],
    )(eo, sel1, sel2)


def _combine_body(y1_ref, y2_ref, g1_ref, g2_ref, y_ref):
    y_ref[...] = g1_ref[...] * y1_ref[...] + g2_ref[...] * y2_ref[...]


def _k5_combine(y1, y2, g1, g2):
    grid = (N // 512,)
    return pl.pallas_call(
        _combine_body,
        grid=grid,
        in_specs=[
            pl.BlockSpec((512, C), lambda i: (i, 0)),
            pl.BlockSpec((512, C), lambda i: (i, 0)),
            pl.BlockSpec((512, 1), lambda i: (i, 0)),
            pl.BlockSpec((512, 1), lambda i: (i, 0)),
        ],
        out_specs=pl.BlockSpec((512, C), lambda i: (i, 0)),
        out_shape=jax.ShapeDtypeStruct((N, C), jnp.float32),
        compiler_params=pltpu.CompilerParams(
            dimension_semantics=("parallel",)),
    )(y1, y2, g1, g2)


def kernel(x, Wr, Wg, Wu, Wp):
    x_flat = x.reshape(N, C)
    sel1, sel2, sl1, sl2, g1, g2, aux = _k1_router(x_flat, Wr)
    tok = _k2a_tokmap(sl1.reshape(N), sl2.reshape(N))
    xd = _k2b_dispatch(x_flat, tok)
    eo = _k3_mlp(xd, Wg, Wu, Wp)
    y1, y2 = _k4_gather(eo, sel1.reshape(N), sel2.reshape(N))
    y = _k5_combine(y1, y2, g1, g2)
    return y.reshape(B, T, C), aux.reshape(())


# speedup vs baseline: 1.1033x; 1.1033x over previous
import functools
import jax
import jax.numpy as jnp
from jax import lax
from jax.experimental import pallas as pl
from jax.experimental.pallas import tpu as pltpu, tpu_sc as plsc

E = 8
TOP_K = 2
CAP_F = 1.25
AUX_COEF = 0.01
B, T, C, I = 2, 2048, 1024, 4096
N = B * T
CAP = 640
NEC = E * CAP
NW = 32
XD_ROWS = 5376
SENT = NEC
BLK = 512
TI = 512

_NEG = -1e30


def _router_body(x_ref, wr_ref, sel1_ref, sel2_ref, sl1_ref, sl2_ref,
                 g1_ref, g2_ref, aux_ref, oh_ref, pos_ref):
    xf = x_ref[...]
    wr = wr_ref[...]
    logits = lax.dot_general(xf, wr, (((1,), (1,)), ((), ())))

    lane = lax.broadcasted_iota(jnp.int32, (N, E), 1)
    m1 = jnp.max(logits, axis=1, keepdims=True)
    eq1 = logits == m1
    idx1 = jnp.min(jnp.where(eq1, lane, E), axis=1, keepdims=True)
    oh1 = (lane == idx1)
    oh1f = oh1.astype(jnp.float32)
    logits2 = jnp.where(oh1, _NEG, logits)
    m2 = jnp.max(logits2, axis=1, keepdims=True)
    eq2 = logits2 == m2
    idx2 = jnp.min(jnp.where(eq2, lane, E), axis=1, keepdims=True)
    oh2 = (lane == idx2)
    oh2f = oh2.astype(jnp.float32)

    ex = jnp.exp(logits - m1)
    sex = jnp.sum(ex, axis=1, keepdims=True)
    probs = ex / sex
    gv1 = jnp.max(probs, axis=1, keepdims=True)
    gv2 = jnp.max(probs - 2.0 * oh1f, axis=1, keepdims=True)
    gsum = jnp.maximum(gv1 + gv2, 1e-9)
    gate1 = gv1 / gsum
    gate2 = gv2 / gsum

    oh_ref[...] = jnp.concatenate([oh1f, oh2f], axis=1)
    r_i = lax.broadcasted_iota(jnp.int32, (BLK, BLK), 0)
    c_i = lax.broadcasted_iota(jnp.int32, (BLK, BLK), 1)
    tri = jnp.where(r_i > c_i, 1.0, 0.0).astype(jnp.float32)

    def body(b, carry):
        blk = oh_ref[pl.ds(b * BLK, BLK), :]
        pos_blk = lax.dot_general(
            tri, blk, (((1,), (0,)), ((), ())),
            precision=lax.Precision.HIGHEST) + carry
        pos_ref[pl.ds(b * BLK, BLK), :] = pos_blk
        return carry + jnp.sum(blk, axis=0, keepdims=True)

    carry = lax.fori_loop(0, N // BLK, body,
                          jnp.zeros((1, 2 * E), jnp.float32))
    posm = pos_ref[...]
    pos1 = jnp.sum(posm[:, :E] * oh1f, axis=1, keepdims=True)
    pos2r = jnp.sum(posm[:, E:] * oh2f, axis=1, keepdims=True)
    tot1 = carry[:, :E]
    tot2 = carry[:, E:]
    count1 = jnp.minimum(tot1, float(CAP))
    pos2 = pos2r + jnp.sum(oh2f * count1, axis=1, keepdims=True)

    mask1 = pos1 < float(CAP)
    mask2 = pos2 < float(CAP)
    slot1 = idx1 * CAP + pos1.astype(jnp.int32)
    slot2 = idx2 * CAP + pos2.astype(jnp.int32)
    sl1_ref[...] = jnp.where(mask1, slot1, SENT)
    sl2_ref[...] = jnp.where(mask2, slot2, SENT)
    sel1_ref[...] = jnp.where(mask1, slot1, 0)
    sel2_ref[...] = jnp.where(mask2, slot2, 0)

    g1m = gate1 * mask1.astype(jnp.float32)
    g2m = gate2 * mask2.astype(jnp.float32)
    denom = jnp.maximum(g1m + g2m, 1e-9)
    g1_ref[...] = g1m / denom
    g2_ref[...] = g2m / denom

    imp = jnp.sum(probs, axis=0, keepdims=True) / float(N)
    loadv = (tot1 + tot2) / float(N * TOP_K)
    aux_ref[...] = jnp.sum(imp * loadv, axis=1, keepdims=True) * (E * AUX_COEF)


def _k1_router(x_flat, wr):
    i32 = lambda: jax.ShapeDtypeStruct((N, 1), jnp.int32)
    f32 = lambda: jax.ShapeDtypeStruct((N, 1), jnp.float32)
    return pl.pallas_call(
        _router_body,
        out_shape=(i32(), i32(), i32(), i32(), f32(), f32(),
                   jax.ShapeDtypeStruct((1, 1), jnp.float32)),
        scratch_shapes=[pltpu.VMEM((N, 2 * E), jnp.float32),
                        pltpu.VMEM((N, 2 * E), jnp.float32)],
    )(x_flat, wr)


def _tokmap_body(sl1_hbm, sl2_hbm, tok_hbm, sl_v, tok_v):
    wid = lax.axis_index("s") * 2 + lax.axis_index("c")

    @pl.when(wid == 0)
    def _():
        def init(i, _):
            tok_v[pl.ds(i * 16, 16)] = jnp.zeros((16,), jnp.int32)
            return 0
        lax.fori_loop(0, XD_ROWS // 16, init, 0)

        pltpu.sync_copy(sl1_hbm, sl_v)

        def scat1(i, _):
            idx = sl_v[pl.ds(i * 16, 16)]
            vals = i * 16 + lax.iota(jnp.int32, 16)
            plsc.store_scatter(tok_v, [idx], vals)
            return 0
        lax.fori_loop(0, N // 16, scat1, 0)

        pltpu.sync_copy(sl2_hbm, sl_v)

        def scat2(i, _):
            idx = sl_v[pl.ds(i * 16, 16)]
            vals = i * 16 + lax.iota(jnp.int32, 16)
            plsc.store_scatter(tok_v, [idx], vals)
            return 0
        lax.fori_loop(0, N // 16, scat2, 0)

        pltpu.sync_copy(tok_v, tok_hbm)


def _k2a_tokmap(sl1, sl2):
    mesh = plsc.VectorSubcoreMesh(core_axis_name="c", subcore_axis_name="s")
    return pl.kernel(
        _tokmap_body,
        out_type=jax.ShapeDtypeStruct((XD_ROWS,), jnp.int32),
        mesh=mesh,
        scratch_types=[pltpu.VMEM((N,), jnp.int32),
                       pltpu.VMEM((XD_ROWS,), jnp.int32)],
        compiler_params=pltpu.CompilerParams(needs_layout_passes=False),
    )(sl1, sl2)


def _dispatch_body(x_hbm, tok_hbm, xd_hbm, idx_v, rows_v, sem):
    wid = lax.axis_index("s") * 2 + lax.axis_index("c")
    base = wid * (XD_ROWS // NW)
    for c in range(3):
        off = base + c * 56
        pltpu.sync_copy(tok_hbm.at[pl.ds(off, 56)], idx_v)
        pltpu.async_copy(x_hbm.at[idx_v], rows_v, sem).wait()
        pltpu.sync_copy(rows_v, xd_hbm.at[pl.ds(off, 56)])


def _k2b_dispatch(x_flat, tok):
    mesh = plsc.VectorSubcoreMesh(core_axis_name="c", subcore_axis_name="s")
    return pl.kernel(
        _dispatch_body,
        out_type=jax.ShapeDtypeStruct((XD_ROWS, C), jnp.float32),
        mesh=mesh,
        scratch_types=[pltpu.VMEM((56,), jnp.int32),
                       pltpu.VMEM((56, C), jnp.float32),
                       pltpu.SemaphoreType.DMA],
    )(x_flat, tok)


def _mlp_body(xd_ref, wg_ref, wu_ref, wp_ref, eo_ref):
    it = pl.program_id(1)
    xb = xd_ref[...]
    wg = wg_ref[0]
    wu = wu_ref[0]
    wp = wp_ref[0]
    g = lax.dot_general(xb, wg, (((1,), (1,)), ((), ())))
    u = lax.dot_general(xb, wu, (((1,), (1,)), ((), ())))
    h = g * (1.0 / (1.0 + jnp.exp(-g))) * u
    contrib = lax.dot_general(h, wp, (((1,), (1,)), ((), ())))

    @pl.when(it == 0)
    def _():
        eo_ref[...] = contrib

    @pl.when(it != 0)
    def _():
        eo_ref[...] += contrib


def _k3_mlp(xd, wg, wu, wp):
    grid = (E, I // TI)
    return pl.pallas_call(
        _mlp_body,
        grid=grid,
        in_specs=[
            pl.BlockSpec((CAP, C), lambda e, it: (e, 0)),
            pl.BlockSpec((1, TI, C), lambda e, it: (e, it, 0)),
            pl.BlockSpec((1, TI, C), lambda e, it: (e, it, 0)),
            pl.BlockSpec((1, C, TI), lambda e, it: (e, 0, it)),
        ],
        out_specs=pl.BlockSpec((CAP, C), lambda e, it: (e, 0)),
        out_shape=jax.ShapeDtypeStruct((NEC, C), jnp.float32),
        compiler_params=pltpu.CompilerParams(
            dimension_semantics=("parallel", "arbitrary")),
    )(xd, wg, wu, wp)


def _gather2_body(eo_hbm, s1_hbm, s2_hbm, y1_hbm, y2_hbm, idx_v, rows_v, sem):
    wid = lax.axis_index("s") * 2 + lax.axis_index("c")
    base = wid * (N // NW)
    for c in range(2):
        off = base + c * 64
        pltpu.sync_copy(s1_hbm.at[pl.ds(off, 64)], idx_v)
        pltpu.async_copy(eo_hbm.at[idx_v], rows_v, sem).wait()
        pltpu.sync_copy(rows_v, y1_hbm.at[pl.ds(off, 64)])
        pltpu.sync_copy(s2_hbm.at[pl.ds(off, 64)], idx_v)
        pltpu.async_copy(eo_hbm.at[idx_v], rows_v, sem).wait()
        pltpu.sync_copy(rows_v, y2_hbm.at[pl.ds(off, 64)])


def _k4_gather(eo, sel1, sel2):
    mesh = plsc.VectorSubcoreMesh(core_axis_name="c", subcore_axis_name="s")
    return pl.kernel(
        _gather2_body,
        out_type=(jax.ShapeDtypeStruct((N, C), jnp.float32),
                  jax.ShapeDtypeStruct((N, C), jnp.float32)),
        mesh=mesh,
        scratch_types=[pltpu.VMEM((64,), jnp.int32),
                       pltpu.VMEM((64, C), jnp.float32),
                       pltpu.SemaphoreType.DMA],
    )(eo, sel1, sel2)


def _combine_body(y1_ref, y2_ref, g1_ref, g2_ref, y_ref):
    y_ref[...] = g1_ref[...] * y1_ref[...] + g2_ref[...] * y2_ref[...]


def _k5_combine(y1, y2, g1, g2):
    grid = (N // 512,)
    return pl.pallas_call(
        _combine_body,
        grid=grid,
        in_specs=[
            pl.BlockSpec((512, C), lambda i: (i, 0)),
            pl.BlockSpec((512, C), lambda i: (i, 0)),
            pl.BlockSpec((512, 1), lambda i: (i, 0)),
            pl.BlockSpec((512, 1), lambda i: (i, 0)),
        ],
        out_specs=pl.BlockSpec((512, C), lambda i: (i, 0)),
        out_shape=jax.ShapeDtypeStruct((N, C), jnp.float32),
        compiler_params=pltpu.CompilerParams(
            dimension_semantics=("parallel",)),
    )(y1, y2, g1, g2)


def kernel(x, Wr, Wg, Wu, Wp):
    x_flat = x.reshape(N, C)
    sel1, sel2, sl1, sl2, g1, g2, aux = _k1_router(x_flat, Wr)
    tok = _k2a_tokmap(sl1.reshape(N), sl2.reshape(N))
    xd = _k2b_dispatch(x_flat, tok)
    eo = _k3_mlp(xd, Wg, Wu, Wp)
    y1, y2 = _k4_gather(eo, sel1.reshape(N), sel2.reshape(N))
    y = _k5_combine(y1, y2, g1, g2)
    return y.reshape(B, T, C), aux.reshape(())


# speedup vs baseline: 1.5851x; 1.4366x over previous
import functools
import jax
import jax.numpy as jnp
from jax import lax
from jax.experimental import pallas as pl
from jax.experimental.pallas import tpu as pltpu, tpu_sc as plsc

E = 8
TOP_K = 2
CAP_F = 1.25
AUX_COEF = 0.01
B, T, C, I = 2, 2048, 1024, 4096
N = B * T
CAP = 640
NEC = E * CAP
NW = 32
XD_ROWS = 5376
SENT = NEC
BLK = 512
TI = 512

_NEG = -1e30


def _router_body(x_ref, wr_ref, sel1_ref, sel2_ref, sl1_ref, sl2_ref,
                 g1_ref, g2_ref, aux_ref, oh_ref, pos_ref):
    xf = x_ref[...]
    wr = wr_ref[...]
    logits = lax.dot_general(xf, wr, (((1,), (1,)), ((), ())))

    lane = lax.broadcasted_iota(jnp.int32, (N, E), 1)
    m1 = jnp.max(logits, axis=1, keepdims=True)
    eq1 = logits == m1
    idx1 = jnp.min(jnp.where(eq1, lane, E), axis=1, keepdims=True)
    oh1 = (lane == idx1)
    oh1f = oh1.astype(jnp.float32)
    logits2 = jnp.where(oh1, _NEG, logits)
    m2 = jnp.max(logits2, axis=1, keepdims=True)
    eq2 = logits2 == m2
    idx2 = jnp.min(jnp.where(eq2, lane, E), axis=1, keepdims=True)
    oh2 = (lane == idx2)
    oh2f = oh2.astype(jnp.float32)

    ex = jnp.exp(logits - m1)
    sex = jnp.sum(ex, axis=1, keepdims=True)
    probs = ex / sex
    gv1 = jnp.max(probs, axis=1, keepdims=True)
    gv2 = jnp.max(probs - 2.0 * oh1f, axis=1, keepdims=True)
    gsum = jnp.maximum(gv1 + gv2, 1e-9)
    gate1 = gv1 / gsum
    gate2 = gv2 / gsum

    oh_ref[...] = jnp.concatenate([oh1f, oh2f], axis=1)
    r_i = lax.broadcasted_iota(jnp.int32, (BLK, BLK), 0)
    c_i = lax.broadcasted_iota(jnp.int32, (BLK, BLK), 1)
    tri = jnp.where(r_i > c_i, 1.0, 0.0).astype(jnp.float32)

    def body(b, carry):
        blk = oh_ref[pl.ds(b * BLK, BLK), :]
        pos_blk = lax.dot_general(
            tri, blk, (((1,), (0,)), ((), ())),
            precision=lax.Precision.HIGHEST) + carry
        pos_ref[pl.ds(b * BLK, BLK), :] = pos_blk
        return carry + jnp.sum(blk, axis=0, keepdims=True)

    carry = lax.fori_loop(0, N // BLK, body,
                          jnp.zeros((1, 2 * E), jnp.float32))
    posm = pos_ref[...]
    pos1 = jnp.sum(posm[:, :E] * oh1f, axis=1, keepdims=True)
    pos2r = jnp.sum(posm[:, E:] * oh2f, axis=1, keepdims=True)
    tot1 = carry[:, :E]
    tot2 = carry[:, E:]
    count1 = jnp.minimum(tot1, float(CAP))
    pos2 = pos2r + jnp.sum(oh2f * count1, axis=1, keepdims=True)

    mask1 = pos1 < float(CAP)
    mask2 = pos2 < float(CAP)
    slot1 = idx1 * CAP + pos1.astype(jnp.int32)
    slot2 = idx2 * CAP + pos2.astype(jnp.int32)
    sl1_ref[...] = jnp.where(mask1, slot1, SENT)
    sl2_ref[...] = jnp.where(mask2, slot2, SENT)
    tok_iota = lax.broadcasted_iota(jnp.int32, (N, 1), 0)
    sel1_ref[...] = jnp.where(mask1, slot1, tok_iota)
    sel2_ref[...] = jnp.where(mask2, slot2, tok_iota)

    g1m = gate1 * mask1.astype(jnp.float32)
    g2m = gate2 * mask2.astype(jnp.float32)
    denom = jnp.maximum(g1m + g2m, 1e-9)
    g1_ref[...] = g1m / denom
    g2_ref[...] = g2m / denom

    imp = jnp.sum(probs, axis=0, keepdims=True) / float(N)
    loadv = (tot1 + tot2) / float(N * TOP_K)
    aux_ref[...] = jnp.sum(imp * loadv, axis=1, keepdims=True) * (E * AUX_COEF)


def _k1_router(x_flat, wr):
    i32 = lambda: jax.ShapeDtypeStruct((N, 1), jnp.int32)
    f32 = lambda: jax.ShapeDtypeStruct((N, 1), jnp.float32)
    return pl.pallas_call(
        _router_body,
        out_shape=(i32(), i32(), i32(), i32(), f32(), f32(),
                   jax.ShapeDtypeStruct((1, 1), jnp.float32)),
        scratch_shapes=[pltpu.VMEM((N, 2 * E), jnp.float32),
                        pltpu.VMEM((N, 2 * E), jnp.float32)],
    )(x_flat, wr)


def _tokmap_body(sl1_hbm, sl2_hbm, tok_hbm, sl_v, tok_v):
    wid = lax.axis_index("s") * 2 + lax.axis_index("c")

    @pl.when(wid == 0)
    def _():
        def init(i, _):
            tok_v[pl.ds(i * 16, 16)] = jnp.zeros((16,), jnp.int32)
            return 0
        lax.fori_loop(0, XD_ROWS // 16, init, 0)

        pltpu.sync_copy(sl1_hbm, sl_v)

        def scat1(i, _):
            idx = sl_v[pl.ds(i * 16, 16)]
            vals = i * 16 + lax.iota(jnp.int32, 16)
            plsc.store_scatter(tok_v, [idx], vals)
            return 0
        lax.fori_loop(0, N // 16, scat1, 0)

        pltpu.sync_copy(sl2_hbm, sl_v)

        def scat2(i, _):
            idx = sl_v[pl.ds(i * 16, 16)]
            vals = i * 16 + lax.iota(jnp.int32, 16)
            plsc.store_scatter(tok_v, [idx], vals)
            return 0
        lax.fori_loop(0, N // 16, scat2, 0)

        pltpu.sync_copy(tok_v, tok_hbm)


def _k2a_tokmap(sl1, sl2):
    mesh = plsc.VectorSubcoreMesh(core_axis_name="c", subcore_axis_name="s")
    return pl.kernel(
        _tokmap_body,
        out_type=jax.ShapeDtypeStruct((XD_ROWS,), jnp.int32),
        mesh=mesh,
        scratch_types=[pltpu.VMEM((N,), jnp.int32),
                       pltpu.VMEM((XD_ROWS,), jnp.int32)],
        compiler_params=pltpu.CompilerParams(needs_layout_passes=False),
    )(sl1, sl2)


def _dispatch_body(x_hbm, tok_hbm, xd_hbm, idx_v, rows_v, sem):
    wid = lax.axis_index("s") * 2 + lax.axis_index("c")
    base = wid * (XD_ROWS // NW)
    for c in range(3):
        off = base + c * 56
        pltpu.sync_copy(tok_hbm.at[pl.ds(off, 56)], idx_v)
        pltpu.async_copy(x_hbm.at[idx_v], rows_v, sem).wait()
        pltpu.sync_copy(rows_v, xd_hbm.at[pl.ds(off, 56)])


def _k2b_dispatch(x_flat, tok):
    mesh = plsc.VectorSubcoreMesh(core_axis_name="c", subcore_axis_name="s")
    return pl.kernel(
        _dispatch_body,
        out_type=jax.ShapeDtypeStruct((XD_ROWS, C), jnp.float32),
        mesh=mesh,
        scratch_types=[pltpu.VMEM((56,), jnp.int32),
                       pltpu.VMEM((56, C), jnp.float32),
                       pltpu.SemaphoreType.DMA],
    )(x_flat, tok)


def _mlp_body(xd_ref, wg_ref, wu_ref, wp_ref, eo_ref):
    it = pl.program_id(1)
    xb = xd_ref[...]
    wg = wg_ref[0]
    wu = wu_ref[0]
    wp = wp_ref[0]
    g = lax.dot_general(xb, wg, (((1,), (1,)), ((), ())))
    u = lax.dot_general(xb, wu, (((1,), (1,)), ((), ())))
    h = g * (1.0 / (1.0 + jnp.exp(-g))) * u
    contrib = lax.dot_general(h, wp, (((1,), (1,)), ((), ())))

    @pl.when(it == 0)
    def _():
        eo_ref[...] = contrib

    @pl.when(it != 0)
    def _():
        eo_ref[...] += contrib


def _k3_mlp(xd, wg, wu, wp):
    grid = (E, I // TI)
    return pl.pallas_call(
        _mlp_body,
        grid=grid,
        in_specs=[
            pl.BlockSpec((CAP, C), lambda e, it: (e, 0)),
            pl.BlockSpec((1, TI, C), lambda e, it: (e, it, 0)),
            pl.BlockSpec((1, TI, C), lambda e, it: (e, it, 0)),
            pl.BlockSpec((1, C, TI), lambda e, it: (e, 0, it)),
        ],
        out_specs=pl.BlockSpec((CAP, C), lambda e, it: (e, 0)),
        out_shape=jax.ShapeDtypeStruct((NEC, C), jnp.float32),
        compiler_params=pltpu.CompilerParams(
            dimension_semantics=("parallel", "arbitrary")),
    )(xd, wg, wu, wp)


def _gather2_body(eo_hbm, s1_hbm, s2_hbm, y1_hbm, y2_hbm, idx_v, rows_v, sem):
    wid = lax.axis_index("s") * 2 + lax.axis_index("c")
    base = wid * (N // NW)
    for c in range(2):
        off = base + c * 64
        pltpu.sync_copy(s1_hbm.at[pl.ds(off, 64)], idx_v)
        pltpu.async_copy(eo_hbm.at[idx_v], rows_v, sem).wait()
        pltpu.sync_copy(rows_v, y1_hbm.at[pl.ds(off, 64)])
        pltpu.sync_copy(s2_hbm.at[pl.ds(off, 64)], idx_v)
        pltpu.async_copy(eo_hbm.at[idx_v], rows_v, sem).wait()
        pltpu.sync_copy(rows_v, y2_hbm.at[pl.ds(off, 64)])


def _k4_gather(eo, sel1, sel2):
    mesh = plsc.VectorSubcoreMesh(core_axis_name="c", subcore_axis_name="s")
    return pl.kernel(
        _gather2_body,
        out_type=(jax.ShapeDtypeStruct((N, C), jnp.float32),
                  jax.ShapeDtypeStruct((N, C), jnp.float32)),
        mesh=mesh,
        scratch_types=[pltpu.VMEM((64,), jnp.int32),
                       pltpu.VMEM((64, C), jnp.float32),
                       pltpu.SemaphoreType.DMA],
    )(eo, sel1, sel2)


def _combine_body(y1_ref, y2_ref, g1_ref, g2_ref, y_ref):
    y_ref[...] = g1_ref[...] * y1_ref[...] + g2_ref[...] * y2_ref[...]


def _k5_combine(y1, y2, g1, g2):
    grid = (N // 512,)
    return pl.pallas_call(
        _combine_body,
        grid=grid,
        in_specs=[
            pl.BlockSpec((512, C), lambda i: (i, 0)),
            pl.BlockSpec((512, C), lambda i: (i, 0)),
            pl.BlockSpec((512, 1), lambda i: (i, 0)),
            pl.BlockSpec((512, 1), lambda i: (i, 0)),
        ],
        out_specs=pl.BlockSpec((512, C), lambda i: (i, 0)),
        out_shape=jax.ShapeDtypeStruct((N, C), jnp.float32),
        compiler_params=pltpu.CompilerParams(
            dimension_semantics=("parallel",)),
    )(y1, y2, g1, g2)


def kernel(x, Wr, Wg, Wu, Wp):
    x_flat = x.reshape(N, C)
    sel1, sel2, sl1, sl2, g1, g2, aux = _k1_router(x_flat, Wr)
    tok = _k2a_tokmap(sl1.reshape(N), sl2.reshape(N))
    xd = _k2b_dispatch(x_flat, tok)
    eo = _k3_mlp(xd, Wg, Wu, Wp)
    y1, y2 = _k4_gather(eo, sel1.reshape(N), sel2.reshape(N))
    y = _k5_combine(y1, y2, g1, g2)
    return y.reshape(B, T, C), aux.reshape(())


# speedup vs baseline: 1.8138x; 1.1443x over previous
import functools
import jax
import jax.numpy as jnp
from jax import lax
from jax.experimental import pallas as pl
from jax.experimental.pallas import tpu as pltpu, tpu_sc as plsc

E = 8
TOP_K = 2
CAP_F = 1.25
AUX_COEF = 0.01
B, T, C, I = 2, 2048, 1024, 4096
N = B * T
CAP = 640
NEC = E * CAP
NW = 32
XD_ROWS = 5376
SENT = NEC
BLK = 512
TI = 1024

_NEG = -1e30


def _pack_bf16_pair(x):
    h = x.shape[1] // 2
    bits = lax.bitcast_convert_type(x, jnp.int32)
    r = bits + 0x7FFF + jnp.bitwise_and(lax.shift_right_logical(bits, 16), 1)
    lo = jnp.bitwise_and(lax.shift_right_logical(r[:, :h], 16), 0xFFFF)
    hi = jnp.bitwise_and(r[:, h:], jnp.int32(-65536))
    return jnp.bitwise_or(lo, hi)


def _unpack_bf16_pair(w):
    lo = lax.bitcast_convert_type(lax.shift_left(w, 16), jnp.float32)
    hi = lax.bitcast_convert_type(jnp.bitwise_and(w, jnp.int32(-65536)),
                                  jnp.float32)
    return jnp.concatenate([lo, hi], axis=1)


def _router_body(x_ref, wr_ref, sel1_ref, sel2_ref, sl1_ref, sl2_ref,
                 g1_ref, g2_ref, aux_ref, x16_ref, oh_ref, pos_ref):
    xf = x_ref[...]
    x16_ref[...] = _pack_bf16_pair(xf)
    wr = wr_ref[...]
    logits = lax.dot_general(xf, wr, (((1,), (1,)), ((), ())))

    lane = lax.broadcasted_iota(jnp.int32, (N, E), 1)
    m1 = jnp.max(logits, axis=1, keepdims=True)
    eq1 = logits == m1
    idx1 = jnp.min(jnp.where(eq1, lane, E), axis=1, keepdims=True)
    oh1 = (lane == idx1)
    oh1f = oh1.astype(jnp.float32)
    logits2 = jnp.where(oh1, _NEG, logits)
    m2 = jnp.max(logits2, axis=1, keepdims=True)
    eq2 = logits2 == m2
    idx2 = jnp.min(jnp.where(eq2, lane, E), axis=1, keepdims=True)
    oh2 = (lane == idx2)
    oh2f = oh2.astype(jnp.float32)

    ex = jnp.exp(logits - m1)
    sex = jnp.sum(ex, axis=1, keepdims=True)
    probs = ex / sex
    gv1 = jnp.max(probs, axis=1, keepdims=True)
    gv2 = jnp.max(probs - 2.0 * oh1f, axis=1, keepdims=True)
    gsum = jnp.maximum(gv1 + gv2, 1e-9)
    gate1 = gv1 / gsum
    gate2 = gv2 / gsum

    oh_ref[...] = jnp.concatenate([oh1f, oh2f], axis=1)
    r_i = lax.broadcasted_iota(jnp.int32, (BLK, BLK), 0)
    c_i = lax.broadcasted_iota(jnp.int32, (BLK, BLK), 1)
    tri = jnp.where(r_i > c_i, 1.0, 0.0).astype(jnp.float32)

    def body(b, carry):
        blk = oh_ref[pl.ds(b * BLK, BLK), :]
        pos_blk = lax.dot_general(
            tri, blk, (((1,), (0,)), ((), ())),
            precision=lax.Precision.HIGHEST) + carry
        pos_ref[pl.ds(b * BLK, BLK), :] = pos_blk
        return carry + jnp.sum(blk, axis=0, keepdims=True)

    carry = lax.fori_loop(0, N // BLK, body,
                          jnp.zeros((1, 2 * E), jnp.float32))
    posm = pos_ref[...]
    pos1 = jnp.sum(posm[:, :E] * oh1f, axis=1, keepdims=True)
    pos2r = jnp.sum(posm[:, E:] * oh2f, axis=1, keepdims=True)
    tot1 = carry[:, :E]
    tot2 = carry[:, E:]
    count1 = jnp.minimum(tot1, float(CAP))
    pos2 = pos2r + jnp.sum(oh2f * count1, axis=1, keepdims=True)

    mask1 = pos1 < float(CAP)
    mask2 = pos2 < float(CAP)
    slot1 = idx1 * CAP + pos1.astype(jnp.int32)
    slot2 = idx2 * CAP + pos2.astype(jnp.int32)
    sl1_ref[...] = jnp.where(mask1, slot1, SENT)
    sl2_ref[...] = jnp.where(mask2, slot2, SENT)
    tok_iota = lax.broadcasted_iota(jnp.int32, (N, 1), 0)
    sel1_ref[...] = jnp.where(mask1, slot1, tok_iota)
    sel2_ref[...] = jnp.where(mask2, slot2, tok_iota)

    g1m = gate1 * mask1.astype(jnp.float32)
    g2m = gate2 * mask2.astype(jnp.float32)
    denom = jnp.maximum(g1m + g2m, 1e-9)
    g1_ref[...] = g1m / denom
    g2_ref[...] = g2m / denom

    imp = jnp.sum(probs, axis=0, keepdims=True) / float(N)
    loadv = (tot1 + tot2) / float(N * TOP_K)
    aux_ref[...] = jnp.sum(imp * loadv, axis=1, keepdims=True) * (E * AUX_COEF)


def _k1_router(x_flat, wr):
    i32 = lambda: jax.ShapeDtypeStruct((N, 1), jnp.int32)
    f32 = lambda: jax.ShapeDtypeStruct((N, 1), jnp.float32)
    return pl.pallas_call(
        _router_body,
        out_shape=(i32(), i32(), i32(), i32(), f32(), f32(),
                   jax.ShapeDtypeStruct((1, 1), jnp.float32),
                   jax.ShapeDtypeStruct((N, C // 2), jnp.int32)),
        scratch_shapes=[pltpu.VMEM((N, 2 * E), jnp.float32),
                        pltpu.VMEM((N, 2 * E), jnp.float32)],
    )(x_flat, wr)


def _dispatch_body(sl1_hbm, sl2_hbm, x_hbm, xd_hbm,
                   sl_v, tok_v, tok_sh, idx_v, rows_v, sem):
    cid = lax.axis_index("c")
    sid = lax.axis_index("s")

    @pl.when(sid == 0)
    def _():
        def init(i, _):
            tok_v[pl.ds(i * 16, 16)] = jnp.zeros((16,), jnp.int32)
            return 0
        lax.fori_loop(0, XD_ROWS // 16, init, 0)

        pltpu.sync_copy(sl1_hbm, sl_v)

        def scat1(i, _):
            idx = sl_v[pl.ds(i * 16, 16)]
            vals = i * 16 + lax.iota(jnp.int32, 16)
            plsc.store_scatter(tok_v, [idx], vals)
            return 0
        lax.fori_loop(0, N // 16, scat1, 0)

        pltpu.sync_copy(sl2_hbm, sl_v)

        def scat2(i, _):
            idx = sl_v[pl.ds(i * 16, 16)]
            vals = i * 16 + lax.iota(jnp.int32, 16)
            plsc.store_scatter(tok_v, [idx], vals)
            return 0
        lax.fori_loop(0, N // 16, scat2, 0)

        pltpu.sync_copy(tok_v, tok_sh)

    plsc.subcore_barrier()
    wid = sid * 2 + cid
    base = wid * (XD_ROWS // NW)
    pltpu.sync_copy(tok_sh.at[pl.ds(base, XD_ROWS // NW)], idx_v)
    pltpu.async_copy(x_hbm.at[idx_v], rows_v, sem).wait()
    pltpu.sync_copy(rows_v, xd_hbm.at[pl.ds(base, XD_ROWS // NW)])


def _k2_dispatch(sl1, sl2, x16):
    mesh = plsc.VectorSubcoreMesh(core_axis_name="c", subcore_axis_name="s")
    return pl.kernel(
        _dispatch_body,
        out_type=jax.ShapeDtypeStruct((XD_ROWS, C // 2), jnp.int32),
        mesh=mesh,
        scratch_types=[pltpu.VMEM((N,), jnp.int32),
                       pltpu.VMEM((XD_ROWS,), jnp.int32),
                       pltpu.VMEM_SHARED((XD_ROWS,), jnp.int32),
                       pltpu.VMEM((XD_ROWS // NW,), jnp.int32),
                       pltpu.VMEM((XD_ROWS // NW, C // 2), jnp.int32),
                       pltpu.SemaphoreType.DMA],
        compiler_params=pltpu.CompilerParams(needs_layout_passes=False),
    )(sl1, sl2, x16)


def _mlp_body(xd_ref, wg_ref, wu_ref, wp_ref, eo_ref, acc_ref):
    it = pl.program_id(1)
    xb = _unpack_bf16_pair(xd_ref[...]).astype(jnp.bfloat16)
    wg = wg_ref[0].astype(jnp.bfloat16)
    wu = wu_ref[0].astype(jnp.bfloat16)
    wp = wp_ref[0].astype(jnp.bfloat16)
    dn = (((1,), (1,)), ((), ()))
    g = lax.dot_general(xb, wg, dn, preferred_element_type=jnp.float32)
    u = lax.dot_general(xb, wu, dn, preferred_element_type=jnp.float32)
    h = (g * (1.0 / (1.0 + jnp.exp(-g))) * u).astype(jnp.bfloat16)
    contrib = lax.dot_general(h, wp, dn, preferred_element_type=jnp.float32)

    @pl.when(it == 0)
    def _():
        acc_ref[...] = contrib

    @pl.when(it != 0)
    def _():
        acc_ref[...] += contrib

    @pl.when(it == I // TI - 1)
    def _():
        eo_ref[...] = _pack_bf16_pair(acc_ref[...])


def _k3_mlp(xd, wg, wu, wp):
    grid = (E, I // TI)
    return pl.pallas_call(
        _mlp_body,
        grid=grid,
        in_specs=[
            pl.BlockSpec((CAP, C // 2), lambda e, it: (e, 0)),
            pl.BlockSpec((1, TI, C), lambda e, it: (e, it, 0)),
            pl.BlockSpec((1, TI, C), lambda e, it: (e, it, 0)),
            pl.BlockSpec((1, C, TI), lambda e, it: (e, 0, it)),
        ],
        out_specs=pl.BlockSpec((CAP, C // 2), lambda e, it: (e, 0)),
        out_shape=jax.ShapeDtypeStruct((NEC, C // 2), jnp.int32),
        scratch_shapes=[pltpu.VMEM((CAP, C), jnp.float32)],
        compiler_params=pltpu.CompilerParams(
            dimension_semantics=("parallel", "arbitrary")),
    )(xd, wg, wu, wp)


def _gather2_body(eo_hbm, s1_hbm, s2_hbm, y1_hbm, y2_hbm,
                  idx0_v, idx1_v, rows0_v, rows1_v,
                  gsem0, gsem1, wsem0, wsem1):
    wid = lax.axis_index("s") * 2 + lax.axis_index("c")
    base = wid * (N // NW)
    units = [(s1_hbm, y1_hbm, base), (s1_hbm, y1_hbm, base + 64),
             (s2_hbm, y2_hbm, base), (s2_hbm, y2_hbm, base + 64)]
    idxs = [idx0_v, idx1_v]
    rows = [rows0_v, rows1_v]
    gsems = [gsem0, gsem1]
    wsems = [wsem0, wsem1]
    gh = [None] * 4
    wb = [None] * 4
    for u, (s_hbm, y_hbm, off) in enumerate(units):
        b = u % 2
        if u >= 2:
            wb[u - 2].wait()
        pltpu.sync_copy(s_hbm.at[pl.ds(off, 64)], idxs[b])
        gh[u] = pltpu.async_copy(eo_hbm.at[idxs[b]], rows[b], gsems[b])
        if u >= 1:
            pb = (u - 1) % 2
            gh[u - 1].wait()
            _, _, poff = units[u - 1]
            wb[u - 1] = pltpu.async_copy(rows[pb], units[u - 1][1].at[pl.ds(poff, 64)], wsems[pb])
    gh[3].wait()
    wb[3] = pltpu.async_copy(rows1_v, y2_hbm.at[pl.ds(base + 64, 64)], wsems[1])
    wb[2].wait()
    wb[3].wait()


def _k4_gather(eo, sel1, sel2):
    mesh = plsc.VectorSubcoreMesh(core_axis_name="c", subcore_axis_name="s")
    return pl.kernel(
        _gather2_body,
        out_type=(jax.ShapeDtypeStruct((N, C // 2), jnp.int32),
                  jax.ShapeDtypeStruct((N, C // 2), jnp.int32)),
        mesh=mesh,
        scratch_types=[pltpu.VMEM((64,), jnp.int32),
                       pltpu.VMEM((64,), jnp.int32),
                       pltpu.VMEM((64, C // 2), jnp.int32),
                       pltpu.VMEM((64, C // 2), jnp.int32),
                       pltpu.SemaphoreType.DMA,
                       pltpu.SemaphoreType.DMA,
                       pltpu.SemaphoreType.DMA,
                       pltpu.SemaphoreType.DMA],
    )(eo, sel1, sel2)


def _combine_body(y1_ref, y2_ref, g1_ref, g2_ref, y_ref):
    a = _unpack_bf16_pair(y1_ref[...])
    b = _unpack_bf16_pair(y2_ref[...])
    y_ref[...] = g1_ref[...] * a + g2_ref[...] * b


def _k5_combine(y1, y2, g1, g2):
    grid = (N // 512,)
    return pl.pallas_call(
        _combine_body,
        grid=grid,
        in_specs=[
            pl.BlockSpec((512, C // 2), lambda i: (i, 0)),
            pl.BlockSpec((512, C // 2), lambda i: (i, 0)),
            pl.BlockSpec((512, 1), lambda i: (i, 0)),
            pl.BlockSpec((512, 1), lambda i: (i, 0)),
        ],
        out_specs=pl.BlockSpec((512, C), lambda i: (i, 0)),
        out_shape=jax.ShapeDtypeStruct((N, C), jnp.float32),
        compiler_params=pltpu.CompilerParams(
            dimension_semantics=("parallel",)),
    )(y1, y2, g1, g2)


def kernel(x, Wr, Wg, Wu, Wp):
    x_flat = x.reshape(N, C)
    sel1, sel2, sl1, sl2, g1, g2, aux, x16 = _k1_router(x_flat, Wr)
    xd = _k2_dispatch(sl1.reshape(N), sl2.reshape(N), x16)
    eo = _k3_mlp(xd, Wg, Wu, Wp)
    y1, y2 = _k4_gather(eo, sel1.reshape(N), sel2.reshape(N))
    y = _k5_combine(y1, y2, g1, g2)
    return y.reshape(B, T, C), aux.reshape(())
